# baseline (speedup 1.0000x reference)
import jax
import jax.numpy as jnp
from jax import lax
from jax.experimental import pallas as pl
from jax.experimental.pallas import tpu as pltpu
from jax.experimental.pallas import tpu_sc as plsc

_B = 16384
_D = 64
_L = 16
_NC = 2
_NS = 16
_NW = _NC * _NS
_ROWS_PER_W = _B // _NW
_CHUNK = 128
_NCHUNK = _ROWS_PER_W // _CHUNK
_CHUNK_ELEMS = _CHUNK * _D

_GATHER_DNUMS = lax.GatherDimensionNumbers(
    offset_dims=(), collapsed_slice_dims=(0,), start_index_map=(0,))


def _permute(v, idx):
    return lax.gather(v, idx[:, None], _GATHER_DNUMS, slice_sizes=(1,),
                      mode=lax.GatherScatterMode.PROMISE_IN_BOUNDS)


def _sc_body(sre_h, sim_h, rre_h, rim_h, ore_h, oim_h, out_h,
             sre_v, sim_v, rre_v, rim_v, ore_v, oim_v, out_v):
    wid = lax.axis_index("s") * _NC + lax.axis_index("c")
    lane = lax.iota(jnp.int32, _L)

    def chunk_body(c, carry):
        base_row = wid * _ROWS_PER_W + c * _CHUNK
        base_el = base_row * _D
        for hbm, vmem in ((sre_h, sre_v), (sim_h, sim_v),
                          (rre_h, rre_v), (rim_h, rim_v),
                          (ore_h, ore_v), (oim_h, oim_v)):
            pltpu.sync_copy(hbm.at[pl.ds(base_el, _CHUNK_ELEMS)], vmem)

        def grp_body(q, qcarry):
            res = jnp.zeros((_L,), jnp.float32)
            for j in range(_L):
                off0 = (q * _L + j) * _D
                acc = None
                for g in range(_D // _L):
                    off = off0 + g * _L
                    vs_re = sre_v[pl.ds(off, _L)]
                    vs_im = sim_v[pl.ds(off, _L)]
                    vr_re = rre_v[pl.ds(off, _L)]
                    vr_im = rim_v[pl.ds(off, _L)]
                    vo_re = ore_v[pl.ds(off, _L)]
                    vo_im = oim_v[pl.ds(off, _L)]
                    u = vs_re * vo_re + vs_im * vo_im
                    w = vs_re * vo_im - vs_im * vo_re
                    t = vr_re * u + vr_im * w
                    acc = t if acc is None else acc + t
                for k in (1, 2, 4, 8):
                    acc = acc + _permute(acc, lane ^ k)
                res = jnp.where(lane == j, acc, res)
            out_v[pl.ds(q * _L, _L)] = res
            return qcarry
        lax.fori_loop(0, _CHUNK // _L, grp_body, 0)

        pltpu.sync_copy(out_v, out_h.at[pl.ds(base_row, _CHUNK)])
        return carry

    lax.fori_loop(0, _NCHUNK, chunk_body, 0)


@jax.jit
def _sc_score(sre, sim, rre, rim, ore, oim):
    mesh = plsc.VectorSubcoreMesh(core_axis_name="c", subcore_axis_name="s")
    f = pl.kernel(
        _sc_body,
        out_type=jax.ShapeDtypeStruct((_B,), jnp.float32),
        mesh=mesh,
        scratch_types=[pltpu.VMEM((_CHUNK_ELEMS,), jnp.float32)
                       for _ in range(6)]
                      + [pltpu.VMEM((_CHUNK,), jnp.float32)],
    )
    return f(sre, sim, rre, rim, ore, oim)


def kernel(s_re, s_im, r_re, r_im, o_re, o_im):
    flat = [x.reshape(-1) for x in (s_re, s_im, r_re, r_im, o_re, o_im)]
    out = _sc_score(*flat)
    return out.reshape(_B, 1)


# speedup vs baseline: 1.2209x; 1.2209x over previous
import jax
import jax.numpy as jnp
from jax import lax
from jax.experimental import pallas as pl
from jax.experimental.pallas import tpu as pltpu
from jax.experimental.pallas import tpu_sc as plsc

_B = 16384
_D = 64
_L = 16
_NC = 2
_NS = 16
_NW = _NC * _NS
_ROWS_PER_W = _B // _NW
_CHUNK = 128
_NCHUNK = _ROWS_PER_W // _CHUNK
_CHUNK_ELEMS = _CHUNK * _D

_GATHER_DNUMS = lax.GatherDimensionNumbers(
    offset_dims=(), collapsed_slice_dims=(0,), start_index_map=(0,))


def _permute(v, idx):
    return lax.gather(v, idx[:, None], _GATHER_DNUMS, slice_sizes=(1,),
                      mode=lax.GatherScatterMode.PROMISE_IN_BOUNDS)


def _sc_body(sre_h, sim_h, rre_h, rim_h, ore_h, oim_h, out_h, *scratch):
    bufs = (scratch[0:6], scratch[6:12])
    outs = scratch[12:14]
    sems = scratch[14:16]
    osems = scratch[16:18]
    hbms = (sre_h, sim_h, rre_h, rim_h, ore_h, oim_h)

    wid = lax.axis_index("s") * _NC + lax.axis_index("c")
    lane = lax.iota(jnp.int32, _L)

    def issue(c):
        b = c % 2
        base_el = (wid * _ROWS_PER_W + c * _CHUNK) * _D
        return [pltpu.async_copy(h.at[pl.ds(base_el, _CHUNK_ELEMS)], v,
                                 sems[b])
                for h, v in zip(hbms, bufs[b])]

    pend = issue(0)
    out_pend = [None, None]
    for c in range(_NCHUNK):
        b = c % 2
        nxt = issue(c + 1) if c + 1 < _NCHUNK else []
        for h in pend:
            h.wait()
        pend = nxt
        if out_pend[b] is not None:
            out_pend[b].wait()
            out_pend[b] = None
        sre_v, sim_v, rre_v, rim_v, ore_v, oim_v = bufs[b]
        out_v = outs[b]

        @plsc.parallel_loop(0, _CHUNK, unroll=2,
                            carry=jnp.zeros((_L,), jnp.float32))
        def row_body(r, res):
            off0 = r * _D
            acc = None
            for g in range(_D // _L):
                off = off0 + g * _L
                vs_re = sre_v[pl.ds(off, _L)]
                vs_im = sim_v[pl.ds(off, _L)]
                vr_re = rre_v[pl.ds(off, _L)]
                vr_im = rim_v[pl.ds(off, _L)]
                vo_re = ore_v[pl.ds(off, _L)]
                vo_im = oim_v[pl.ds(off, _L)]
                u = vs_re * vo_re + vs_im * vo_im
                w = vs_re * vo_im - vs_im * vo_re
                t = vr_re * u + vr_im * w
                acc = t if acc is None else acc + t
            for k in (8, 4, 2, 1):
                acc = acc + _permute(acc, lane ^ k)
            j = jnp.bitwise_and(r, _L - 1)
            res = jnp.where(lane == j, acc, res)
            done = j == _L - 1

            @pl.when(done)
            def _():
                out_v[pl.ds(r - (_L - 1), _L)] = res

            return jnp.where(done, jnp.zeros((_L,), jnp.float32), res)

        base_row = wid * _ROWS_PER_W + c * _CHUNK
        out_pend[b] = pltpu.async_copy(
            out_v.at[pl.ds(0, _CHUNK)], out_h.at[pl.ds(base_row, _CHUNK)],
            osems[b])
    for h in out_pend:
        if h is not None:
            h.wait()


@jax.jit
def _sc_score(sre, sim, rre, rim, ore, oim):
    mesh = plsc.VectorSubcoreMesh(core_axis_name="c", subcore_axis_name="s")
    f = pl.kernel(
        _sc_body,
        out_type=jax.ShapeDtypeStruct((_B,), jnp.float32),
        mesh=mesh,
        scratch_types=[pltpu.VMEM((_CHUNK_ELEMS,), jnp.float32)
                       for _ in range(12)]
                      + [pltpu.VMEM((_CHUNK + _L,), jnp.float32)
                         for _ in range(2)]
                      + [pltpu.SemaphoreType.DMA for _ in range(4)],
    )
    return f(sre, sim, rre, rim, ore, oim)


def kernel(s_re, s_im, r_re, r_im, o_re, o_im):
    flat = [x.reshape(-1) for x in (s_re, s_im, r_re, r_im, o_re, o_im)]
    out = _sc_score(*flat)
    return out.reshape(_B, 1)


# speedup vs baseline: 4.8964x; 4.0104x over previous
import jax
import jax.numpy as jnp
from jax import lax
from jax.experimental import pallas as pl
from jax.experimental.pallas import tpu as pltpu
from jax.experimental.pallas import tpu_sc as plsc

_B = 16384
_D = 64
_L = 16
_NC = 2
_NS = 16
_NW = _NC * _NS

_NB_SC = 4096
_B_TC = _B - _NB_SC
_SC_ROWS = _NB_SC // _NW


def _sc_body(sre_h, sim_h, rre_h, rim_h, ore_h, oim_h, out_h, *scratch):
    bufs = scratch[0:6]
    out_v = scratch[6]
    sem, osem = scratch[7], scratch[8]
    hbms = (sre_h, sim_h, rre_h, rim_h, ore_h, oim_h)

    wid = lax.axis_index("s") * _NC + lax.axis_index("c")
    base_col = _B_TC + wid * _SC_ROWS

    pend = [pltpu.async_copy(h.at[:, pl.ds(base_col, _SC_ROWS)], v, sem)
            for h, v in zip(hbms, bufs)]
    for h in pend:
        h.wait()
    sre_v, sim_v, rre_v, rim_v, ore_v, oim_v = bufs

    @plsc.parallel_loop(0, _SC_ROWS // _L, unroll=1)
    def grp_body(j):
        col = j * _L

        def d_body(d4, accs):
            new = []
            for q in range(4):
                d = d4 * 4 + q
                vs_re = sre_v[d, pl.ds(col, _L)]
                vs_im = sim_v[d, pl.ds(col, _L)]
                vr_re = rre_v[d, pl.ds(col, _L)]
                vr_im = rim_v[d, pl.ds(col, _L)]
                vo_re = ore_v[d, pl.ds(col, _L)]
                vo_im = oim_v[d, pl.ds(col, _L)]
                u = vs_re * vo_re + vs_im * vo_im
                w = vs_re * vo_im - vs_im * vo_re
                new.append(accs[q] + vr_re * u + vr_im * w)
            return tuple(new)

        accs = lax.fori_loop(
            0, _D // 4, d_body,
            tuple(jnp.zeros((_L,), jnp.float32) for _ in range(4)))
        out_v[pl.ds(col, _L)] = (accs[0] + accs[1]) + (accs[2] + accs[3])

    pltpu.async_copy(out_v, out_h.at[pl.ds(wid * _SC_ROWS, _SC_ROWS)],
                     osem).wait()


def _sc_score(sre, sim, rre, rim, ore, oim):
    mesh = plsc.VectorSubcoreMesh(core_axis_name="c", subcore_axis_name="s")
    f = pl.kernel(
        _sc_body,
        out_type=jax.ShapeDtypeStruct((_NB_SC,), jnp.float32),
        mesh=mesh,
        scratch_types=[pltpu.VMEM((_D, _SC_ROWS), jnp.float32)
                       for _ in range(6)]
                      + [pltpu.VMEM((_SC_ROWS,), jnp.float32)]
                      + [pltpu.SemaphoreType.DMA for _ in range(2)],
    )
    return f(sre, sim, rre, rim, ore, oim)


_TCC = 3072
_TCN = _B_TC // _TCC


def _tc_block(sre, sim, rre, rim, ore, oim, out):
    u = sre[...] * ore[...] + sim[...] * oim[...]
    w = sre[...] * oim[...] - sim[...] * ore[...]
    combo = rre[...] * u + rim[...] * w
    out[...] = jnp.sum(combo, axis=0, keepdims=True)


def _tc_score(sre, sim, rre, rim, ore, oim):
    in_spec = pl.BlockSpec((_D, _TCC), lambda i: (0, i))
    return pl.pallas_call(
        _tc_block,
        grid=(_TCN,),
        in_specs=[in_spec] * 6,
        out_specs=pl.BlockSpec((1, _TCC), lambda i: (0, i)),
        out_shape=jax.ShapeDtypeStruct((1, _B_TC), jnp.float32),
        compiler_params=pltpu.CompilerParams(
            dimension_semantics=("parallel",)),
    )(sre, sim, rre, rim, ore, oim)


@jax.jit
def _score(*cols):
    sc_out = _sc_score(*cols)
    tc_out = _tc_score(*cols)
    return jnp.concatenate([tc_out.reshape(-1), sc_out])


def kernel(s_re, s_im, r_re, r_im, o_re, o_im):
    cols = [jnp.squeeze(x, 1).T
            for x in (s_re, s_im, r_re, r_im, o_re, o_im)]
    return _score(*cols).reshape(_B, 1)
